# baseline (speedup 1.0000x reference)
import functools

import jax
import jax.numpy as jnp
from jax import lax
from jax.experimental import pallas as pl
from jax.experimental.pallas import tpu as pltpu
from jax.experimental.pallas import tpu_sc as plsc

NUM_HEAD = 8
R0, R1 = 2, 2
WX = 2 * R0 + 2
WY = 2 * R1 + 2
A = WX * WY
AP = 40
NW = 32
CH = 64
_INTERPRET = False


def _mm_kernel(x_ref, w_ref, o_ref):
    o_ref[...] = jnp.dot(x_ref[...], w_ref[...],
                         preferred_element_type=jnp.float32)


def _mm(x2d, w, interpret):
    M, K = x2d.shape
    Nn = w.shape[1]
    MB = next(m for m in (1152, 512, 256, 128, 64, 32, 16, 8) if M % m == 0)
    return pl.pallas_call(
        _mm_kernel,
        out_shape=jax.ShapeDtypeStruct((M, Nn), jnp.float32),
        grid=(M // MB,),
        in_specs=[
            pl.BlockSpec((MB, K), lambda m: (m, 0)),
            pl.BlockSpec((K, Nn), lambda m: (0, 0)),
        ],
        out_specs=pl.BlockSpec((MB, Nn), lambda m: (m, 0)),
        interpret=interpret,
    )(x2d, w)


def _rsum(v):
    return plsc.cumsum(v)[15]


def _attn_sc_body(q_hbm, kv_hbm, idx_hbm, fw_hbm, bias_hbm, out_hbm,
                  qv, idxv, fwv, biasv, outv, kvb0, kvb1, sem0, sem1,
                  *, BN, C):
    wid = lax.axis_index("s") * 2 + lax.axis_index("c")
    QW = BN // NW
    base0 = wid * QW

    pltpu.sync_copy(bias_hbm, biasv)

    def chunk_body(ch, _):
        base = base0 + ch * CH
        pltpu.sync_copy(q_hbm.at[pl.ds(base, CH)], qv)
        pltpu.sync_copy(idx_hbm.at[pl.ds(base, CH)], idxv)
        pltpu.sync_copy(fw_hbm.at[pl.ds(base, CH)], fwv)

        def one_query(j, kvb):
            zv = jnp.zeros((16,), jnp.float32)
            iot = lax.iota(jnp.int32, 16)
            perms = [iot ^ sh for sh in (8, 4, 2, 1)]

            def rsum(v):
                for p in perms:
                    v = v + v[p]
                return v

            fwrow = fwv[j, pl.ds(0, 16)]
            for h in range(NUM_HEAD):
                hq = h * 48
                qh = [qv[j, pl.ds(hq + c * 16, 16)] for c in range(3)]
                sv = [zv, zv, zv]
                for a in range(A):
                    s = rsum(kvb[a, pl.ds(hq, 16)] * qh[0] +
                             kvb[a, pl.ds(hq + 16, 16)] * qh[1] +
                             kvb[a, pl.ds(hq + 32, 16)] * qh[2])
                    g = a // 16
                    sv[g] = jnp.where(iot == (a % 16), s, sv[g])

                wacc = [zv, zv, zv]
                for s_idx in range(4):
                    e = [jnp.exp(sv[g] + biasv[s_idx, pl.ds(g * 16, 16)])
                         for g in range(3)]
                    ssum = rsum(e[0] + e[1] + e[2])
                    wf = (zv + fwrow[s_idx]) / ssum
                    wacc = [wacc[g] + wf * e[g] for g in range(3)]

                acc = [zv, zv, zv]
                for a in range(A):
                    w = wacc[a // 16][a % 16]
                    acc = [acc[c] + w * kvb[a, pl.ds(C + hq + c * 16, 16)]
                           for c in range(3)]
                for c in range(3):
                    outv[j, pl.ds(hq + c * 16, 16)] = acc[c]

        def q1_body(j, _):
            pltpu.async_copy(kv_hbm.at[idxv.at[j]], kvb0, sem0).wait()
            one_query(j, kvb0)
            return 0

        lax.fori_loop(0, CH, q1_body, 0)
        pltpu.sync_copy(outv, out_hbm.at[pl.ds(base, CH)])
        return 0

    lax.fori_loop(0, QW // CH, chunk_body, 0)


def kernel(x, max_offset, Wq, Wk, Wv, Wproj):
    Bb, Hh, Ww, C = x.shape
    HEAD_DIM = C // NUM_HEAD
    SCALE = HEAD_DIM ** -0.5
    N = Hh * Ww
    BN = Bb * N

    x2d = x.reshape(BN, C)
    wcat = jnp.concatenate([Wq.T * SCALE, Wk.T, Wv.T], axis=1)
    qkv = _mm(x2d, wcat, _INTERPRET)
    q2 = qkv[:, :C]
    kvt = qkv[:, C:]

    mo = max_offset.reshape(Bb, N, 2)
    ox = jnp.clip(mo[..., 0], R0, Ww - 1 - R0 - 0.001)
    oy = jnp.clip(mo[..., 1], R1, Hh - 1 - R1 - 0.001)
    mxf = jnp.floor(ox)
    myf = jnp.floor(oy)
    fx = ox - mxf
    fy = oy - myf
    mx = mxf.astype(jnp.int32)
    my = myf.astype(jnp.int32)
    dys = jnp.arange(-R1, R1 + 2)
    dxs = jnp.arange(-R0, R0 + 2)
    pos = ((my[..., None, None] + dys[:, None]) * Ww +
           (mx[..., None, None] + dxs[None, :]))
    pos = pos + (jnp.arange(Bb) * N)[:, None, None, None]
    idxt = pos.reshape(BN, A)
    idxt = jnp.concatenate(
        [idxt, jnp.zeros((BN, AP - A), jnp.int32)], axis=1)

    fw = jnp.stack([(1 - fy) * (1 - fx), (1 - fy) * fx,
                    fy * (1 - fx), fy * fx], axis=-1).reshape(BN, 4)
    fw = jnp.concatenate([fw, jnp.zeros((BN, 12), jnp.float32)], axis=1)

    la = jnp.arange(48)[None, :]
    ldy = la // WX
    ldx = la % WX
    biases = []
    for sy, sx in ((0, 0), (0, 1), (1, 0), (1, 1)):
        m = ((la < A) & (ldy >= sy) & (ldy <= sy + 2 * R1) &
             (ldx >= sx) & (ldx <= sx + 2 * R0))
        biases.append(jnp.where(m, 0.0, -1e30).astype(jnp.float32))
    biasc = jnp.stack(biases, axis=0).reshape(4, 48)

    mesh = plsc.VectorSubcoreMesh(core_axis_name="c", subcore_axis_name="s")
    attn = functools.partial(
        pl.kernel,
        out_type=jax.ShapeDtypeStruct((BN, C), jnp.float32),
        mesh=mesh,
        scratch_types=[
            pltpu.VMEM((CH, C), jnp.float32),
            pltpu.VMEM((CH, AP), jnp.int32),
            pltpu.VMEM((CH, 16), jnp.float32),
            pltpu.VMEM((4, 48), jnp.float32),
            pltpu.VMEM((CH, C), jnp.float32),
            pltpu.VMEM((AP, 2 * C), jnp.float32),
            pltpu.VMEM((AP, 2 * C), jnp.float32),
            pltpu.SemaphoreType.DMA(()),
            pltpu.SemaphoreType.DMA(()),
        ],
    )(functools.partial(_attn_sc_body, BN=BN, C=C))
    agg = attn(q2, kvt, idxt, fw, biasc)

    y = _mm(agg, Wproj.T, _INTERPRET)
    return y.reshape(Bb, Hh, Ww, C)


# speedup vs baseline: 2.2088x; 2.2088x over previous
import functools

import jax
import jax.numpy as jnp
from jax.experimental import pallas as pl
from jax.experimental.pallas import tpu as pltpu

NUM_HEAD = 8
R0, R1 = 2, 2
WX = 2 * R0 + 2
WY = 2 * R1 + 2
TQ = 64
SLOT = 8 * WY


def _mm_kernel(x_ref, w_ref, o_ref):
    o_ref[...] = jnp.dot(x_ref[...], w_ref[...],
                         preferred_element_type=jnp.float32)


def _mm(x2d, w):
    M, K = x2d.shape
    Nn = w.shape[1]
    MB = next(m for m in (1152, 512, 256, 128, 64, 32, 16, 8) if M % m == 0)
    return pl.pallas_call(
        _mm_kernel,
        out_shape=jax.ShapeDtypeStruct((M, Nn), jnp.float32),
        grid=(M // MB,),
        in_specs=[
            pl.BlockSpec((MB, K), lambda m: (m, 0)),
            pl.BlockSpec((K, Nn), lambda m: (0, 0)),
        ],
        out_specs=pl.BlockSpec((MB, Nn), lambda m: (m, 0)),
    )(x2d, w)


def _attn_kernel(pb_ref, q_ref, kvh_ref, fx_ref, fy_ref, hm_ref, bias_ref,
                 o_ref, kvg_s, kv_vm, sem, *, Ww, C):
    b = pl.program_id(0)
    i = pl.program_id(1)

    @pl.when(jnp.logical_and(b == 0, i == 0))
    def _init():
        kvg_s[...] = jnp.zeros_like(kvg_s)

    @pl.when(i == 0)
    def _stage_kv():
        cp = pltpu.make_async_copy(kvh_ref.at[b], kv_vm, sem)
        cp.start()
        cp.wait()

    def gather_one(iq, _):
        p = pb_ref[0, 0, iq]
        ph = (p // 8) * 8
        shift = 16 - (p - ph)
        for dy in range(WY):
            src = pl.ds(ph + dy * Ww, 16)
            dst = pl.ds(iq * SLOT + dy * 8, WX)
            a = pltpu.roll(kv_vm[src, :], shift, 0)
            kvg_s[dst, :] = a[:WX, :]
        return 0

    jax.lax.fori_loop(0, TQ, gather_one, 0, unroll=8)

    prod3 = (kvg_s[:, :C].reshape(TQ, SLOT, C) * q_ref[0][:, None, :])
    scores = jnp.dot(prod3.reshape(TQ * SLOT, C), hm_ref[...],
                     preferred_element_type=jnp.float32)

    st = jnp.swapaxes(scores.reshape(TQ, SLOT, NUM_HEAD), 1, 2)
    X = st.reshape(TQ * NUM_HEAD, SLOT)

    fx = jnp.broadcast_to(fx_ref[0][:, None, :],
                          (TQ, NUM_HEAD, 1)).reshape(TQ * NUM_HEAD, 1)
    fy = jnp.broadcast_to(fy_ref[0][:, None, :],
                          (TQ, NUM_HEAD, 1)).reshape(TQ * NUM_HEAD, 1)
    W_acc = jnp.zeros_like(X)
    for s_idx, (sy, sx) in enumerate(((0, 0), (0, 1), (1, 0), (1, 1))):
        e = jnp.exp(X + bias_ref[s_idx:s_idx + 1, :])
        ssum = jnp.sum(e, axis=-1, keepdims=True)
        wy = fy if sy else (1.0 - fy)
        wx = fx if sx else (1.0 - fx)
        W_acc = W_acc + (wy * wx / ssum) * e

    Wb = jnp.swapaxes(W_acc.reshape(TQ, NUM_HEAD, SLOT), 1, 2).reshape(
        TQ * SLOT, NUM_HEAD)
    broad = jnp.dot(Wb, hm_ref[...].T, preferred_element_type=jnp.float32)
    weighted = broad * kvg_s[:, C:]
    o_ref[0] = jnp.sum(weighted.reshape(TQ, SLOT, C), axis=1)


def kernel(x, max_offset, Wq, Wk, Wv, Wproj):
    Bb, Hh, Ww, C = x.shape
    HEAD_DIM = C // NUM_HEAD
    SCALE = HEAD_DIM ** -0.5
    N = Hh * Ww
    NB = N // TQ

    x2d = x.reshape(Bb * N, C)
    wcat = jnp.concatenate([Wq.T * SCALE, Wk.T, Wv.T], axis=1)
    qkv = _mm(x2d, wcat)
    q = qkv[:, :C].reshape(Bb, N, C)
    kv = qkv[:, C:].reshape(Bb, N, 2 * C)

    mo = max_offset.reshape(Bb, N, 2)
    ox = jnp.clip(mo[..., 0], R0, Ww - 1 - R0 - 0.001)
    oy = jnp.clip(mo[..., 1], R1, Hh - 1 - R1 - 0.001)
    mxf = jnp.floor(ox)
    myf = jnp.floor(oy)
    fx = (ox - mxf).reshape(Bb, N, 1)
    fy = (oy - myf).reshape(Bb, N, 1)
    pbase = ((myf.astype(jnp.int32) - R1) * Ww +
             (mxf.astype(jnp.int32) - R0)).reshape(Bb * NB, 1, TQ)

    dl = jnp.arange(C)[:, None]
    hm = (dl // HEAD_DIM == jnp.arange(NUM_HEAD)[None, :]).astype(jnp.float32)
    ldy = jnp.arange(SLOT)[None, :] // 8
    ldx = jnp.arange(SLOT)[None, :] % 8
    biases = []
    for sy, sx in ((0, 0), (0, 1), (1, 0), (1, 1)):
        m = ((ldy >= sy) & (ldy <= sy + 2 * R1) &
             (ldx >= sx) & (ldx <= sx + 2 * R0))
        biases.append(jnp.where(m, 0.0, -1e30).astype(jnp.float32))
    bias = jnp.concatenate(biases + biases, axis=0)

    kvp = jnp.concatenate([kv, jnp.zeros((Bb, 16, 2 * C), jnp.float32)],
                          axis=1)
    agg = pl.pallas_call(
        functools.partial(_attn_kernel, Ww=Ww, C=C),
        out_shape=jax.ShapeDtypeStruct((Bb, N, C), jnp.float32),
        grid=(Bb, NB),
        in_specs=[
            pl.BlockSpec((1, 1, TQ), lambda b, i, NB=NB: (b * NB + i, 0, 0),
                         memory_space=pltpu.SMEM),
            pl.BlockSpec((1, TQ, C), lambda b, i: (b, i, 0)),
            pl.BlockSpec(memory_space=pltpu.MemorySpace.HBM),
            pl.BlockSpec((1, TQ, 1), lambda b, i: (b, i, 0)),
            pl.BlockSpec((1, TQ, 1), lambda b, i: (b, i, 0)),
            pl.BlockSpec((C, NUM_HEAD), lambda b, i: (0, 0)),
            pl.BlockSpec((8, SLOT), lambda b, i: (0, 0)),
        ],
        out_specs=pl.BlockSpec((1, TQ, C), lambda b, i: (b, i, 0)),
        scratch_shapes=[
            pltpu.VMEM((TQ * SLOT, 2 * C), jnp.float32),
            pltpu.VMEM((N + 16, 2 * C), jnp.float32),
            pltpu.SemaphoreType.DMA,
        ],
    )(pbase, q, kvp, fx, fy, hm, bias)

    y = _mm(agg.reshape(Bb * N, C), Wproj.T)
    return y.reshape(Bb, Hh, Ww, C)
